# baseline (speedup 1.0000x reference)
import functools

import jax
import jax.numpy as jnp
from jax import lax
from jax.experimental import pallas as pl
from jax.experimental.pallas import tpu as pltpu
from jax.experimental.pallas import tpu_sc as plsc

_BATCH = 16384
_FEAT = 64
_NC = 2
_NS = 16
_NW = _NC * _NS
_BPW = _BATCH // _NW
_CH = _BPW // 128
_LANES = 16
_FCH = _FEAT // _LANES


def _make_kernel():
    mesh = plsc.VectorSubcoreMesh(core_axis_name="c", subcore_axis_name="s")

    @functools.partial(
        pl.kernel,
        mesh=mesh,
        out_type=jax.ShapeDtypeStruct((_NW, _LANES), jnp.float32),
        compiler_params=pltpu.CompilerParams(use_tc_tiling_on_sc=False),
        scratch_types=[
            pltpu.VMEM((_CH, 128), jnp.int32),
            pltpu.VMEM((_BPW, _FEAT), jnp.float32),
            pltpu.VMEM((_BPW, _FEAT), jnp.float32),
            pltpu.VMEM((_LANES,), jnp.float32),
            pltpu.SemaphoreType.DMA,
            pltpu.SemaphoreType.DMA,
        ],
    )
    def sc_center_loss(feats_hbm, labels_hbm, centers_hbm, out_hbm,
                       idx_v, feat_v, cent_v, out_v, gsem, fsem):
        wid = lax.axis_index("s") * _NC + lax.axis_index("c")

        pltpu.sync_copy(labels_hbm.at[pl.ds(wid * _CH, _CH)], idx_v)
        fcopy = pltpu.async_copy(
            feats_hbm.at[pl.ds(wid * _BPW, _BPW)], feat_v, fsem)
        gathers = [
            pltpu.async_copy(
                centers_hbm.at[idx_v.at[j]],
                cent_v.at[pl.ds(j * 128, 128)], gsem)
            for j in range(_CH)
        ]
        for g in gathers:
            g.wait()
        fcopy.wait()

        def body(r, accs):
            new = []
            for j in range(_FCH):
                f = feat_v[r, pl.ds(j * _LANES, _LANES)]
                c = cent_v[r, pl.ds(j * _LANES, _LANES)]
                d = f - c
                new.append(accs[j] + d * d)
            return tuple(new)

        zero = jnp.zeros((_LANES,), jnp.float32)
        accs = lax.fori_loop(0, _BPW, body, (zero,) * _FCH)
        out_v[...] = (accs[0] + accs[1]) + (accs[2] + accs[3])
        pltpu.sync_copy(out_v, out_hbm.at[wid])

    return sc_center_loss


_sc_center_loss = _make_kernel()


def kernel(feats, labels, centers):
    labels2d = labels.astype(jnp.int32).reshape(_BATCH // 128, 128)
    partials = _sc_center_loss(feats, labels2d, centers)
    return jnp.sum(partials) * (1.0 / _BATCH)


# speedup vs baseline: 1.3126x; 1.3126x over previous
import functools

import jax
import jax.numpy as jnp
from jax import lax
from jax.experimental import pallas as pl
from jax.experimental.pallas import tpu as pltpu
from jax.experimental.pallas import tpu_sc as plsc

_BATCH = 16384
_FEAT = 64
_NC = 2
_NS = 16
_NW = _NC * _NS
_BPW = _BATCH // _NW
_CHUNK = 128
_NCHUNK = _BPW // _CHUNK
_NBUF = 3
_LANES = 16
_FCH = _FEAT // _LANES


def _make_kernel():
    mesh = plsc.VectorSubcoreMesh(core_axis_name="c", subcore_axis_name="s")

    @functools.partial(
        pl.kernel,
        mesh=mesh,
        out_type=jax.ShapeDtypeStruct((_NW, _LANES), jnp.float32),
        compiler_params=pltpu.CompilerParams(use_tc_tiling_on_sc=True),
        scratch_types=[
            pltpu.VMEM((_BPW,), jnp.int32),
            pltpu.VMEM((_BPW, _FEAT), jnp.float32),
            pltpu.VMEM((_NBUF, _CHUNK, _FEAT), jnp.float32),
            pltpu.VMEM((_LANES,), jnp.float32),
            pltpu.SemaphoreType.DMA,
            pltpu.SemaphoreType.DMA,
            pltpu.SemaphoreType.DMA,
            pltpu.SemaphoreType.DMA,
        ],
    )
    def sc_center_loss(feats_hbm, labels_hbm, centers_hbm, out_hbm,
                       idx_v, feat_v, cent_v, out_v, fsem, g0, g1, g2):
        gsems = (g0, g1, g2)
        wid = lax.axis_index("s") * _NC + lax.axis_index("c")
        base = wid * _BPW

        pltpu.sync_copy(labels_hbm.at[pl.ds(base, _BPW)], idx_v)
        fcopy = pltpu.async_copy(
            feats_hbm.at[pl.ds(base, _BPW)], feat_v, fsem)

        def fire_chunk(c, buf):
            def fire_group(g, carry):
                idx_vec = idx_v[pl.ds(c * _CHUNK + g * _LANES, _LANES)]
                for lane in range(_LANES):
                    r = idx_vec[lane]
                    pltpu.async_copy(
                        centers_hbm.at[pl.ds(r, 1)],
                        cent_v.at[buf, pl.ds(g * _LANES + lane, 1)],
                        gsems[buf])
                return carry
            lax.fori_loop(0, _CHUNK // _LANES, fire_group, 0)

        def drain_chunk(buf):
            pltpu.make_async_copy(
                centers_hbm.at[pl.ds(0, _CHUNK)],
                cent_v.at[buf], gsems[buf]).wait()

        for b in range(_NBUF):
            fire_chunk(b, b)
        fcopy.wait()

        def row_body(c, buf):
            def body(i, accs):
                new = []
                for j in range(_FCH):
                    f = feat_v[c * _CHUNK + i, pl.ds(j * _LANES, _LANES)]
                    g = cent_v[buf, i, pl.ds(j * _LANES, _LANES)]
                    d = f - g
                    new.append(accs[j] + d * d)
                return tuple(new)
            return body

        zero = jnp.zeros((_LANES,), jnp.float32)
        accs = (zero,) * _FCH
        for c in range(_NCHUNK):
            buf = c % _NBUF
            drain_chunk(buf)
            accs = lax.fori_loop(0, _CHUNK, row_body(c, buf), accs)
            if c + _NBUF < _NCHUNK:
                fire_chunk(c + _NBUF, buf)

        out_v[...] = (accs[0] + accs[1]) + (accs[2] + accs[3])
        pltpu.sync_copy(out_v, out_hbm.at[wid])

    return sc_center_loss


_sc_center_loss = _make_kernel()


def kernel(feats, labels, centers):
    partials = _sc_center_loss(feats, labels.astype(jnp.int32), centers)
    return jnp.sum(partials) * (1.0 / _BATCH)
